# Initial kernel scaffold; baseline (speedup 1.0000x reference)
import functools

import jax
import jax.numpy as jnp
from jax import lax
from jax.experimental import pallas as pl
from jax.experimental.pallas import tpu as pltpu
from jax.experimental.pallas import tpu_sc as plsc

NC = 2
NS = 16
NW = NC * NS
LANES = 16
CHUNK = 128

HID = 64


def _rup(a, m):
    return (a + m - 1) // m * m


def _sc_mesh():
    return plsc.VectorSubcoreMesh(
        core_axis_name="c", subcore_axis_name="s", num_cores=NC,
        num_subcores=NS)


@functools.partial(jax.jit, static_argnames=("e_pad",))
def _edge_gather(x16, send, recv, *, e_pad):
    epw = e_pad // NW
    nchunk = epw // CHUNK

    @functools.partial(
        pl.kernel,
        out_type=(
            jax.ShapeDtypeStruct((e_pad, 16), jnp.float32),
            jax.ShapeDtypeStruct((e_pad, 16), jnp.float32),
        ),
        mesh=_sc_mesh(),
        scratch_types=[
            pltpu.VMEM((CHUNK,), jnp.int32),
            pltpu.VMEM((CHUNK,), jnp.int32),
            pltpu.VMEM((CHUNK, 16), jnp.float32),
            pltpu.VMEM((CHUNK, 16), jnp.float32),
            pltpu.SemaphoreType.DMA,
            pltpu.SemaphoreType.DMA,
        ],
    )
    def k(x_hbm, send_hbm, recv_hbm, xs_hbm, xr_hbm, si_v, ri_v, sr_v, rr_v,
          sem0, sem1):
        w = lax.axis_index("s") * NC + lax.axis_index("c")
        base = w * epw

        def chunk(ci, _):
            off = base + ci * CHUNK
            pltpu.sync_copy(send_hbm.at[pl.ds(off, CHUNK)], si_v)
            pltpu.sync_copy(recv_hbm.at[pl.ds(off, CHUNK)], ri_v)
            cp0 = pltpu.async_copy(x_hbm.at[si_v], sr_v, sem0)
            cp1 = pltpu.async_copy(x_hbm.at[ri_v], rr_v, sem1)
            cp0.wait()
            cp1.wait()
            pltpu.sync_copy(sr_v, xs_hbm.at[pl.ds(off, CHUNK)])
            pltpu.sync_copy(rr_v, xr_hbm.at[pl.ds(off, CHUNK)])
            return 0

        lax.fori_loop(0, nchunk, chunk, 0)

    return k(x16, send, recv)


@functools.partial(jax.jit, static_argnames=("n_pad", "e_pad", "passes",
                                             "gather_src"))
def _seg_scatter(src, recv, dst, zsrc, *, n_pad, e_pad, passes, gather_src):
    n_half = n_pad // 2
    zslice = _rup((n_half + 16) // NS + 1, 64)
    acc_rows = NS * zslice
    trash = n_half
    fslice = n_half // NS
    ept = e_pad // NS
    nchunk = ept // CHUNK
    p_tot = passes

    @functools.partial(
        pl.kernel,
        out_type=jax.ShapeDtypeStruct((n_pad, HID * p_tot), jnp.float32),
        mesh=_sc_mesh(),
        scratch_types=[
            pltpu.VMEM((CHUNK,), jnp.int32),
            pltpu.VMEM((CHUNK,), jnp.int32),
            pltpu.VMEM((CHUNK,), jnp.int32),
            pltpu.VMEM((CHUNK, HID), jnp.float32),
            pltpu.VMEM_SHARED((acc_rows, HID), jnp.float32),
            pltpu.SemaphoreType.DMA,
        ],
    )
    def k(src_hbm, recv_hbm, dst_hbm, z_hbm, out_hbm, rv_v, gi_v, dl_v,
          rows_v, acc, sem):
        c = lax.axis_index("c")
        s = lax.axis_index("s")
        node_base = c * n_half
        ebase = s * ept

        for p in range(p_tot):
            pltpu.sync_copy(z_hbm, acc.at[pl.ds(s * zslice, zslice)])
            plsc.subcore_barrier()

            def chunk(ci, _):
                off = ebase + ci * CHUNK
                pltpu.sync_copy(dst_hbm.at[pl.ds(off, CHUNK)], dl_v)
                if gather_src:
                    pltpu.sync_copy(recv_hbm.at[pl.ds(off, CHUNK)], rv_v)
                for j in range(CHUNK // LANES):
                    sl = pl.ds(j * LANES, LANES)
                    dv = dl_v[sl] - node_base
                    ok = (dv >= 0) & (dv < n_half)
                    dl_v[sl] = jnp.where(ok, dv, trash)
                    if gather_src:
                        if p_tot > 1:
                            gi_v[sl] = rv_v[sl] * p_tot + p
                        else:
                            gi_v[sl] = rv_v[sl]
                if gather_src:
                    pltpu.async_copy(src_hbm.at[gi_v], rows_v, sem).wait()
                else:
                    pltpu.sync_copy(src_hbm.at[pl.ds(off, CHUNK)], rows_v)
                pltpu.sync_copy(rows_v, acc.at[dl_v], add=True)
                return 0

            lax.fori_loop(0, nchunk, chunk, 0)
            plsc.subcore_barrier()
            if p_tot > 1:
                pltpu.sync_copy(
                    acc.at[pl.ds(s * fslice, fslice)],
                    out_hbm.at[pl.ds(node_base + s * fslice, fslice),
                               pl.ds(p * HID, HID)])
            else:
                pltpu.sync_copy(
                    acc.at[pl.ds(s * fslice, fslice)],
                    out_hbm.at[pl.ds(node_base + s * fslice, fslice)])
            plsc.subcore_barrier()

    return k(src, recv, dst, zsrc)


def _edge_mlp(xs, xr, send, recv, w0, b0, w1, b1, w2, b2, wout, bout, n_real,
              block=2048):
    e_pad = xs.shape[0]
    grid = e_pad // block

    def body(xs_r, xr_r, send_r, recv_r, w0_r, b0_r, w1_r, b1_r, w2_r, b2_r,
             wout_r, bout_r, msg_r, sendk_r):
        xs_b = xs_r[...]
        xr_b = xr_r[...]
        diff = xr_b - xs_b
        d3 = diff[:, 0:3]
        dist2 = jnp.sum(d3 * d3, axis=1, keepdims=True)
        dist = jnp.sqrt(dist2)
        iszero = dist == 0.0
        safe = jnp.where(iszero, 1.0, dist)
        vects = jnp.where(iszero, 0.0, d3 / safe)
        zcol = jnp.zeros_like(dist)
        e8 = jnp.concatenate([diff[:, 3:6], dist, vects, zcol], axis=1)
        kn = jax.nn.relu(
            jnp.dot(e8, w0_r[...], preferred_element_type=jnp.float32)
            + b0_r[...][None, :])
        kn = jax.nn.relu(
            jnp.dot(kn, w1_r[...], preferred_element_type=jnp.float32)
            + b1_r[...][None, :])
        kn = jax.nn.relu(
            jnp.dot(kn, w2_r[...], preferred_element_type=jnp.float32)
            + b2_r[...][None, :])
        kflat = (jnp.dot(kn, wout_r[...], preferred_element_type=jnp.float32)
                 + bout_r[...][None, :])
        keep = xs_b[:, 3] <= xr_b[:, 3]
        valid = recv_r[...] < n_real
        kf = jnp.logical_and(keep, valid)
        msg = jnp.zeros((block, HID), jnp.float32)
        for f in range(6):
            msg = msg + xs_b[:, f:f + 1] * kflat[:, f * HID:(f + 1) * HID]
        msg_r[...] = msg * kf.astype(jnp.float32)[:, None]
        sendk_r[...] = jnp.where(kf, send_r[...], jnp.int32(2 ** 27))

    fullspec = [pl.BlockSpec(w.shape, lambda k, nd=w.ndim: (0,) * nd)
                for w in (w0, b0, w1, b1, w2, b2, wout, bout)]
    return pl.pallas_call(
        body,
        grid=(grid,),
        in_specs=[
            pl.BlockSpec((block, 16), lambda k: (k, 0)),
            pl.BlockSpec((block, 16), lambda k: (k, 0)),
            pl.BlockSpec((block,), lambda k: (k,)),
            pl.BlockSpec((block,), lambda k: (k,)),
            *fullspec,
        ],
        out_specs=[
            pl.BlockSpec((block, HID), lambda k: (k, 0)),
            pl.BlockSpec((block,), lambda k: (k,)),
        ],
        out_shape=[
            jax.ShapeDtypeStruct((e_pad, HID), jnp.float32),
            jax.ShapeDtypeStruct((e_pad,), jnp.int32),
        ],
    )(xs, xr, send, recv, w0, b0, w1, b1, w2, b2, wout, bout)


def _node_linear(g, h, w1, w2, b, block=1792):
    n_pad = g.shape[0]
    grid = n_pad // block
    cout = w1.shape[1]

    def body(g_r, h_r, w1_r, w2_r, b_r, o_r):
        acc = jnp.dot(g_r[...], w1_r[...], preferred_element_type=jnp.float32)
        acc = acc + jnp.dot(h_r[...], w2_r[...],
                            preferred_element_type=jnp.float32)
        o_r[...] = jax.nn.relu(acc + b_r[...][None, :])

    return pl.pallas_call(
        body,
        grid=(grid,),
        in_specs=[
            pl.BlockSpec((block, g.shape[1]), lambda k: (k, 0)),
            pl.BlockSpec((block, h.shape[1]), lambda k: (k, 0)),
            pl.BlockSpec(w1.shape, lambda k: (0, 0)),
            pl.BlockSpec(w2.shape, lambda k: (0, 0)),
            pl.BlockSpec(b.shape, lambda k: (0,)),
        ],
        out_specs=pl.BlockSpec((block, cout), lambda k: (k, 0)),
        out_shape=jax.ShapeDtypeStruct((n_pad, cout), jnp.float32),
    )(g, h, w1, w2, b)


def _pool_decode(h, ids, dws, dbs, dscales, dshifts, w_fin, b_fin, nseg,
                 block=1792):
    n_pad = h.shape[0]
    grid = n_pad // block
    width = h.shape[1]
    n_dec = len(dws)

    def body(h_r, ids_r, *refs):
        dw_r = refs[0:n_dec]
        db_r = refs[n_dec:2 * n_dec]
        dsc_r = refs[2 * n_dec:3 * n_dec]
        dsh_r = refs[3 * n_dec:4 * n_dec]
        wf_r = refs[4 * n_dec]
        bf_r = refs[4 * n_dec + 1]
        o_r = refs[4 * n_dec + 2]
        psum_s = refs[4 * n_dec + 3]
        pmax_s = refs[4 * n_dec + 4]
        cnt_s = refs[4 * n_dec + 5]

        step = pl.program_id(0)

        @pl.when(step == 0)
        def _init():
            psum_s[...] = jnp.zeros_like(psum_s)
            pmax_s[...] = jnp.full_like(pmax_s, -jnp.inf)
            cnt_s[...] = jnp.zeros_like(cnt_s)

        h_b = h_r[...]
        ids_b = ids_r[...]
        oh = (ids_b[:, None]
              == lax.broadcasted_iota(jnp.int32, (block, nseg), 1)
              ).astype(jnp.float32)
        psum_s[...] += jnp.dot(oh.T, h_b, preferred_element_type=jnp.float32)
        cnt_s[...] += jnp.sum(oh, axis=0, keepdims=True)

        lo = jnp.min(ids_b)
        hi = jnp.minimum(jnp.max(ids_b), nseg - 1)

        def seg(b, _):
            mask = (ids_b == b)[:, None]
            colmax = jnp.max(jnp.where(mask, h_b, -jnp.inf), axis=0,
                             keepdims=True)
            pmax_s[pl.ds(b, 1), :] = jnp.maximum(pmax_s[pl.ds(b, 1), :],
                                                 colmax)
            return 0

        lax.fori_loop(lo, hi + 1, seg, 0)

        @pl.when(step == grid - 1)
        def _decode():
            cnt = jnp.maximum(cnt_s[...], 1.0)
            pavg = psum_s[...] / cnt.T
            z = jnp.concatenate([pmax_s[...], pavg, psum_s[...]], axis=1)
            for li in range(n_dec):
                z = jnp.dot(z, dw_r[li][...],
                            preferred_element_type=jnp.float32)
                z = z + db_r[li][...][None, :]
                z = jnp.where(z > 0, z, 0.1 * z)
                z = z * dsc_r[li][...][None, :] + dsh_r[li][...][None, :]
            o_r[...] = (jnp.dot(z, wf_r[...],
                                preferred_element_type=jnp.float32)
                        + bf_r[...][None, :])

    def full(a):
        return pl.BlockSpec(a.shape, lambda k, nd=a.ndim: (0,) * nd)

    return pl.pallas_call(
        body,
        grid=(grid,),
        in_specs=[
            pl.BlockSpec((block, width), lambda k: (k, 0)),
            pl.BlockSpec((block,), lambda k: (k,)),
            *[full(w) for w in dws],
            *[full(b) for b in dbs],
            *[full(s) for s in dscales],
            *[full(s) for s in dshifts],
            full(w_fin), full(b_fin),
        ],
        out_specs=pl.BlockSpec((nseg, 128), lambda k: (0, 0)),
        out_shape=jax.ShapeDtypeStruct((nseg, 128), jnp.float32),
        scratch_shapes=[
            pltpu.VMEM((nseg, width), jnp.float32),
            pltpu.VMEM((nseg, width), jnp.float32),
            pltpu.VMEM((1, nseg), jnp.float32),
        ],
    )(h, ids, *dws, *dbs, *dscales, *dshifts, w_fin, b_fin)


def kernel(x, edge_index, i, params):
    p = params
    n = x.shape[0]
    e = edge_index.shape[1]
    nseg = 128
    n_half = _rup(_rup(n, 2) // 2, 128)
    n_pad = 2 * n_half
    e_pad = _rup(e, CHUNK * NW)

    x16 = jnp.zeros((n_pad, 16), jnp.float32).at[:n, :6].set(x)
    send = jnp.concatenate(
        [edge_index[0].astype(jnp.int32),
         jnp.zeros((e_pad - e,), jnp.int32)])
    recv = jnp.concatenate(
        [edge_index[1].astype(jnp.int32),
         jnp.full((e_pad - e,), n, jnp.int32)])
    ids = jnp.concatenate(
        [i.astype(jnp.int32), jnp.full((n_pad - n,), nseg, jnp.int32)])
    zslice = _rup((n_half + 16) // NS + 1, 64)
    zsrc = jnp.zeros((zslice, HID), jnp.float32)

    w0 = jnp.zeros((8, HID), jnp.float32).at[:7].set(p['ecc_kn_W0'])
    root16 = jnp.zeros((16, HID), jnp.float32).at[:6].set(p['ecc_root'])
    eye64 = jnp.eye(HID, dtype=jnp.float32)

    xs, xr = _edge_gather(x16, send, recv, e_pad=e_pad)

    msg, sendk = _edge_mlp(
        xs, xr, send, recv, w0, p['ecc_kn_b0'], p['ecc_kn_W1'],
        p['ecc_kn_b1'], p['ecc_kn_W2'], p['ecc_kn_b2'], p['ecc_kn_Wout'],
        p['ecc_kn_bout'], n)

    h_pre = _seg_scatter(msg, recv, recv, zsrc, n_pad=n_pad, e_pad=e_pad,
                         passes=1, gather_src=False)

    h = _node_linear(h_pre, x16, eye64, root16, p['ecc_bias'])

    for li in range(4):
        w1 = p['gcs%d_W1' % li]
        cin = w1.shape[0]
        np_ = cin // HID
        h2 = h.reshape(n_pad * np_, HID)
        g = _seg_scatter(h2, recv, sendk, zsrc, n_pad=n_pad, e_pad=e_pad,
                         passes=np_, gather_src=True)
        h = _node_linear(g, h, w1, p['gcs%d_W2' % li], p['gcs%d_b' % li])

    dws, dbs, dscales, dshifts = [], [], [], []
    for li in range(5):
        dws.append(p['dec%d_W' % li])
        dbs.append(p['dec%d_b' % li])
        sc = p['bn%d_gamma' % li] * lax.rsqrt(p['bn%d_mv' % li] + 1e-3)
        dscales.append(sc)
        dshifts.append(p['bn%d_beta' % li] - p['bn%d_mm' % li] * sc)
    w_fin = jnp.zeros((dws[-1].shape[1], 128), jnp.float32
                      ).at[:, :7].set(p['d2_W'])
    b_fin = jnp.zeros((128,), jnp.float32).at[:7].set(p['d2_b'])

    out = _pool_decode(h, ids, dws, dbs, dscales, dshifts, w_fin, b_fin,
                       nseg)
    return out[:, :7]


# speedup vs baseline: 3.4552x; 3.4552x over previous
import functools

import jax
import jax.numpy as jnp
from jax import lax
from jax.experimental import pallas as pl
from jax.experimental.pallas import tpu as pltpu
from jax.experimental.pallas import tpu_sc as plsc

NC = 2
NS = 16
NW = NC * NS
LANES = 16
CHUNK = 128

HID = 64


def _rup(a, m):
    return (a + m - 1) // m * m


def _sc_mesh():
    return plsc.VectorSubcoreMesh(
        core_axis_name="c", subcore_axis_name="s", num_cores=NC,
        num_subcores=NS)


_SC_PARAMS = pltpu.CompilerParams(use_tc_tiling_on_sc=False)


@functools.partial(jax.jit, static_argnames=("e_pad",))
def _edge_gather(x16, send, recv, *, e_pad):
    epw = e_pad // NW
    nchunk = epw // CHUNK

    @functools.partial(
        pl.kernel,
        out_type=(
            jax.ShapeDtypeStruct((e_pad, 16), jnp.float32),
            jax.ShapeDtypeStruct((e_pad, 16), jnp.float32),
        ),
        mesh=_sc_mesh(),
        compiler_params=_SC_PARAMS,
        scratch_types=[
            pltpu.VMEM((CHUNK,), jnp.int32),
            pltpu.VMEM((CHUNK,), jnp.int32),
            pltpu.VMEM((CHUNK, 16), jnp.float32),
            pltpu.VMEM((CHUNK, 16), jnp.float32),
            pltpu.SemaphoreType.DMA,
            pltpu.SemaphoreType.DMA,
        ],
    )
    def k(x_hbm, send_hbm, recv_hbm, xs_hbm, xr_hbm, si_v, ri_v, sr_v, rr_v,
          sem0, sem1):
        w = lax.axis_index("s") * NC + lax.axis_index("c")
        base = w * epw

        def chunk(ci, _):
            off = base + ci * CHUNK
            pltpu.sync_copy(send_hbm.at[pl.ds(off, CHUNK)], si_v)
            pltpu.sync_copy(recv_hbm.at[pl.ds(off, CHUNK)], ri_v)
            cp0 = pltpu.async_copy(x_hbm.at[si_v], sr_v, sem0)
            cp1 = pltpu.async_copy(x_hbm.at[ri_v], rr_v, sem1)
            cp0.wait()
            cp1.wait()
            pltpu.sync_copy(sr_v, xs_hbm.at[pl.ds(off, CHUNK)])
            pltpu.sync_copy(rr_v, xr_hbm.at[pl.ds(off, CHUNK)])
            return 0

        lax.fori_loop(0, nchunk, chunk, 0)

    return k(x16, send, recv)


@functools.partial(jax.jit, static_argnames=("n_pad", "e_pad", "passes",
                                             "gather_src"))
def _seg_scatter(src, recv, dst, zsrc, *, n_pad, e_pad, passes, gather_src):
    n_half = n_pad // 2
    zslice = _rup((n_half + 16) // NS + 1, 64)
    acc_rows = NS * zslice
    trash = n_half
    fslice = n_half // NS
    ept = e_pad // NS
    nchunk = ept // CHUNK
    p_tot = passes

    @functools.partial(
        pl.kernel,
        out_type=jax.ShapeDtypeStruct((n_pad, HID * p_tot), jnp.float32),
        mesh=_sc_mesh(),
        compiler_params=_SC_PARAMS,
        scratch_types=[
            pltpu.VMEM((CHUNK,), jnp.int32),
            pltpu.VMEM((CHUNK,), jnp.int32),
            pltpu.VMEM((CHUNK,), jnp.int32),
            pltpu.VMEM((CHUNK, HID), jnp.float32),
            pltpu.VMEM_SHARED((acc_rows, HID), jnp.float32),
            pltpu.SemaphoreType.DMA,
        ],
    )
    def k(src_hbm, recv_hbm, dst_hbm, z_hbm, out_hbm, rv_v, gi_v, dl_v,
          rows_v, acc, sem):
        c = lax.axis_index("c")
        s = lax.axis_index("s")
        node_base = c * n_half
        ebase = s * ept

        for p in range(p_tot):
            pltpu.sync_copy(z_hbm, acc.at[pl.ds(s * zslice, zslice)])
            plsc.subcore_barrier()

            def chunk(ci, _):
                off = ebase + ci * CHUNK
                pltpu.sync_copy(dst_hbm.at[pl.ds(off, CHUNK)], dl_v)
                if gather_src:
                    pltpu.sync_copy(recv_hbm.at[pl.ds(off, CHUNK)], rv_v)
                for j in range(CHUNK // LANES):
                    sl = pl.ds(j * LANES, LANES)
                    dv = dl_v[sl] - node_base
                    ok = (dv >= 0) & (dv < n_half)
                    dl_v[sl] = jnp.where(ok, dv, trash)
                    if gather_src:
                        if p_tot > 1:
                            gi_v[sl] = rv_v[sl] * p_tot + p
                        else:
                            gi_v[sl] = rv_v[sl]
                if gather_src:
                    pltpu.async_copy(src_hbm.at[gi_v], rows_v, sem).wait()
                else:
                    pltpu.sync_copy(src_hbm.at[pl.ds(off, CHUNK)], rows_v)
                pltpu.sync_copy(rows_v, acc.at[dl_v], add=True)
                return 0

            lax.fori_loop(0, nchunk, chunk, 0)
            plsc.subcore_barrier()
            if p_tot > 1:
                pltpu.sync_copy(
                    acc.at[pl.ds(s * fslice, fslice)],
                    out_hbm.at[pl.ds(node_base + s * fslice, fslice),
                               pl.ds(p * HID, HID)])
            else:
                pltpu.sync_copy(
                    acc.at[pl.ds(s * fslice, fslice)],
                    out_hbm.at[pl.ds(node_base + s * fslice, fslice)])
            plsc.subcore_barrier()

    return k(src, recv, dst, zsrc)


def _edge_mlp(xs, xr, send, recv, w0, b0, w1, b1, w2, b2, wout, bout, n_real,
              block=2048):
    e_pad = xs.shape[0]
    grid = e_pad // block

    def body(xs_r, xr_r, send_r, recv_r, w0_r, b0_r, w1_r, b1_r, w2_r, b2_r,
             wout_r, bout_r, msg_r, sendk_r):
        xs_b = xs_r[...]
        xr_b = xr_r[...]
        diff = xr_b - xs_b
        d3 = diff[:, 0:3]
        dist2 = jnp.sum(d3 * d3, axis=1, keepdims=True)
        dist = jnp.sqrt(dist2)
        iszero = dist == 0.0
        safe = jnp.where(iszero, 1.0, dist)
        vects = jnp.where(iszero, 0.0, d3 / safe)
        zcol = jnp.zeros_like(dist)
        e8 = jnp.concatenate([diff[:, 3:6], dist, vects, zcol], axis=1)
        kn = jax.nn.relu(
            jnp.dot(e8, w0_r[...], preferred_element_type=jnp.float32)
            + b0_r[...][None, :])
        kn = jax.nn.relu(
            jnp.dot(kn, w1_r[...], preferred_element_type=jnp.float32)
            + b1_r[...][None, :])
        kn = jax.nn.relu(
            jnp.dot(kn, w2_r[...], preferred_element_type=jnp.float32)
            + b2_r[...][None, :])
        kflat = (jnp.dot(kn, wout_r[...], preferred_element_type=jnp.float32)
                 + bout_r[...][None, :])
        keep = xs_b[:, 3] <= xr_b[:, 3]
        valid = recv_r[...] < n_real
        kf = jnp.logical_and(keep, valid)
        msg = jnp.zeros((block, HID), jnp.float32)
        for f in range(6):
            msg = msg + xs_b[:, f:f + 1] * kflat[:, f * HID:(f + 1) * HID]
        msg_r[...] = msg * kf.astype(jnp.float32)[:, None]
        sendk_r[...] = jnp.where(kf, send_r[...], jnp.int32(2 ** 27))

    fullspec = [pl.BlockSpec(w.shape, lambda k, nd=w.ndim: (0,) * nd)
                for w in (w0, b0, w1, b1, w2, b2, wout, bout)]
    return pl.pallas_call(
        body,
        grid=(grid,),
        in_specs=[
            pl.BlockSpec((block, 16), lambda k: (k, 0)),
            pl.BlockSpec((block, 16), lambda k: (k, 0)),
            pl.BlockSpec((block,), lambda k: (k,)),
            pl.BlockSpec((block,), lambda k: (k,)),
            *fullspec,
        ],
        out_specs=[
            pl.BlockSpec((block, HID), lambda k: (k, 0)),
            pl.BlockSpec((block,), lambda k: (k,)),
        ],
        out_shape=[
            jax.ShapeDtypeStruct((e_pad, HID), jnp.float32),
            jax.ShapeDtypeStruct((e_pad,), jnp.int32),
        ],
    )(xs, xr, send, recv, w0, b0, w1, b1, w2, b2, wout, bout)


def _node_linear(g, h, w1, w2, b, block=1792):
    n_pad = g.shape[0]
    grid = n_pad // block
    cout = w1.shape[1]

    def body(g_r, h_r, w1_r, w2_r, b_r, o_r):
        acc = jnp.dot(g_r[...], w1_r[...], preferred_element_type=jnp.float32)
        acc = acc + jnp.dot(h_r[...], w2_r[...],
                            preferred_element_type=jnp.float32)
        o_r[...] = jax.nn.relu(acc + b_r[...][None, :])

    return pl.pallas_call(
        body,
        grid=(grid,),
        in_specs=[
            pl.BlockSpec((block, g.shape[1]), lambda k: (k, 0)),
            pl.BlockSpec((block, h.shape[1]), lambda k: (k, 0)),
            pl.BlockSpec(w1.shape, lambda k: (0, 0)),
            pl.BlockSpec(w2.shape, lambda k: (0, 0)),
            pl.BlockSpec(b.shape, lambda k: (0,)),
        ],
        out_specs=pl.BlockSpec((block, cout), lambda k: (k, 0)),
        out_shape=jax.ShapeDtypeStruct((n_pad, cout), jnp.float32),
    )(g, h, w1, w2, b)


def _pool_decode(h, ids, dws, dbs, dscales, dshifts, w_fin, b_fin, nseg,
                 block=1024):
    n_pad = h.shape[0]
    grid = n_pad // block
    width = h.shape[1]
    n_dec = len(dws)

    def body(h_r, ids_r, *refs):
        dw_r = refs[0:n_dec]
        db_r = refs[n_dec:2 * n_dec]
        dsc_r = refs[2 * n_dec:3 * n_dec]
        dsh_r = refs[3 * n_dec:4 * n_dec]
        wf_r = refs[4 * n_dec]
        bf_r = refs[4 * n_dec + 1]
        o_r = refs[4 * n_dec + 2]
        psum_s = refs[4 * n_dec + 3]
        pmax_s = refs[4 * n_dec + 4]
        cnt_s = refs[4 * n_dec + 5]

        step = pl.program_id(0)

        @pl.when(step == 0)
        def _init():
            psum_s[...] = jnp.zeros_like(psum_s)
            pmax_s[...] = jnp.full_like(pmax_s, -jnp.inf)
            cnt_s[...] = jnp.zeros_like(cnt_s)

        h_b = h_r[...]
        ids_b = ids_r[...]
        ids2 = ids_b[:, None]
        oh = (ids2 == lax.broadcasted_iota(jnp.int32, (block, nseg), 1)
              ).astype(jnp.float32)
        psum_s[...] += jnp.dot(oh.T, h_b, preferred_element_type=jnp.float32)
        cnt_s[...] += jnp.sum(oh, axis=0, keepdims=True)

        lo = jnp.min(ids_b)
        hi = jnp.minimum(jnp.max(ids_b), nseg - 1)

        def seg(b, _):
            mask = ids2 == b
            colmax = jnp.max(jnp.where(mask, h_b, -jnp.inf), axis=0,
                             keepdims=True)
            pmax_s[pl.ds(b, 1), :] = jnp.maximum(pmax_s[pl.ds(b, 1), :],
                                                 colmax)
            return 0

        lax.fori_loop(lo, hi + 1, seg, 0)

        @pl.when(step == grid - 1)
        def _decode():
            cnt = jnp.maximum(cnt_s[...], 1.0)
            pavg = psum_s[...] / cnt.T
            z = jnp.concatenate([pmax_s[...], pavg, psum_s[...]], axis=1)
            for li in range(n_dec):
                z = jnp.dot(z, dw_r[li][...],
                            preferred_element_type=jnp.float32)
                z = z + db_r[li][...][None, :]
                z = jnp.where(z > 0, z, 0.1 * z)
                z = z * dsc_r[li][...][None, :] + dsh_r[li][...][None, :]
            o_r[...] = (jnp.dot(z, wf_r[...],
                                preferred_element_type=jnp.float32)
                        + bf_r[...][None, :])

    def full(a):
        return pl.BlockSpec(a.shape, lambda k, nd=a.ndim: (0,) * nd)

    return pl.pallas_call(
        body,
        grid=(grid,),
        in_specs=[
            pl.BlockSpec((block, width), lambda k: (k, 0)),
            pl.BlockSpec((block,), lambda k: (k,)),
            *[full(w) for w in dws],
            *[full(b) for b in dbs],
            *[full(s) for s in dscales],
            *[full(s) for s in dshifts],
            full(w_fin), full(b_fin),
        ],
        out_specs=pl.BlockSpec((nseg, 128), lambda k: (0, 0)),
        out_shape=jax.ShapeDtypeStruct((nseg, 128), jnp.float32),
        scratch_shapes=[
            pltpu.VMEM((nseg, width), jnp.float32),
            pltpu.VMEM((nseg, width), jnp.float32),
            pltpu.VMEM((1, nseg), jnp.float32),
        ],
    )(h, ids, *dws, *dbs, *dscales, *dshifts, w_fin, b_fin)


def kernel(x, edge_index, i, params):
    p = params
    n = x.shape[0]
    e = edge_index.shape[1]
    nseg = 128
    n_half = _rup(_rup(n, 2) // 2, 128)
    n_pad = 2 * n_half
    e_pad = _rup(e, CHUNK * NW)

    x16 = jnp.zeros((n_pad, 16), jnp.float32).at[:n, :6].set(x)
    send = jnp.concatenate(
        [edge_index[0].astype(jnp.int32),
         jnp.zeros((e_pad - e,), jnp.int32)])
    recv = jnp.concatenate(
        [edge_index[1].astype(jnp.int32),
         jnp.full((e_pad - e,), n, jnp.int32)])
    ids = jnp.concatenate(
        [i.astype(jnp.int32), jnp.full((n_pad - n,), nseg, jnp.int32)])
    zslice = _rup((n_half + 16) // NS + 1, 64)
    zsrc = jnp.zeros((zslice, HID), jnp.float32)

    w0 = jnp.zeros((8, HID), jnp.float32).at[:7].set(p['ecc_kn_W0'])
    root16 = jnp.zeros((16, HID), jnp.float32).at[:6].set(p['ecc_root'])
    eye64 = jnp.eye(HID, dtype=jnp.float32)

    xs, xr = _edge_gather(x16, send, recv, e_pad=e_pad)

    msg, sendk = _edge_mlp(
        xs, xr, send, recv, w0, p['ecc_kn_b0'], p['ecc_kn_W1'],
        p['ecc_kn_b1'], p['ecc_kn_W2'], p['ecc_kn_b2'], p['ecc_kn_Wout'],
        p['ecc_kn_bout'], n)

    h_pre = _seg_scatter(msg, recv, recv, zsrc, n_pad=n_pad, e_pad=e_pad,
                         passes=1, gather_src=False)

    h = _node_linear(h_pre, x16, eye64, root16, p['ecc_bias'])

    for li in range(4):
        w1 = p['gcs%d_W1' % li]
        cin = w1.shape[0]
        np_ = cin // HID
        h2 = h.reshape(n_pad * np_, HID)
        g = _seg_scatter(h2, recv, sendk, zsrc, n_pad=n_pad, e_pad=e_pad,
                         passes=np_, gather_src=True)
        h = _node_linear(g, h, w1, p['gcs%d_W2' % li], p['gcs%d_b' % li])

    dws, dbs, dscales, dshifts = [], [], [], []
    for li in range(5):
        dws.append(p['dec%d_W' % li])
        dbs.append(p['dec%d_b' % li])
        sc = p['bn%d_gamma' % li] * lax.rsqrt(p['bn%d_mv' % li] + 1e-3)
        dscales.append(sc)
        dshifts.append(p['bn%d_beta' % li] - p['bn%d_mm' % li] * sc)
    w_fin = jnp.zeros((dws[-1].shape[1], 128), jnp.float32
                      ).at[:, :7].set(p['d2_W'])
    b_fin = jnp.zeros((128,), jnp.float32).at[:7].set(p['d2_b'])

    out = _pool_decode(h, ids, dws, dbs, dscales, dshifts, w_fin, b_fin,
                       nseg)
    return out[:, :7]
